# Initial kernel scaffold; baseline (speedup 1.0000x reference)
import jax
import jax.numpy as jnp
from jax.experimental import pallas as pl


def kernel(seq, nedge, edgeattr, pedge, emb_table, Wm1, bm1, Ws1, bs1, Wm2, bm2, Ws2, bs2, W_out, b_out, W_e, b_e, W_n, b_n):
    raise NotImplementedError("write your pallas kernel here")


# speedup vs baseline: 3.4545x; 3.4545x over previous
import jax
import jax.numpy as jnp
from jax import lax
from jax.experimental import pallas as pl
from jax.experimental.pallas import tpu as pltpu
from jax.experimental.pallas import tpu_sc as plsc

N = 10000
D = 128
NUM_TYPES = 64
E = 320000
P = 100000

NC = 2
NS = 16
NW = NC * NS
CK = 128

EC = -(-E // (NW * CK))
E_PAD = NW * CK * EC
PC = -(-P // (NW * CK))
P_PAD = NW * CK * PC
NPAD = 10240
RT = NPAD // NS

BN = 2000
HIGH = lax.Precision.HIGHEST

_MESH = plsc.VectorSubcoreMesh(
    core_axis_name="c", subcore_axis_name="s", num_cores=NC, num_subcores=NS)


def _dot(a, b):
    return jnp.dot(a, b, preferred_element_type=jnp.float32, precision=HIGH)


def _q(v):
    return v.astype(jnp.bfloat16).astype(jnp.float32)


def _embed_body(seq_ref, emb_ref, o_ref):
    s = seq_ref[...]
    oh = (s == lax.broadcasted_iota(jnp.int32, (BN, NUM_TYPES), 1))
    o_ref[...] = _q(_dot(oh.astype(jnp.float32), emb_ref[...]))


def _embed(seq_col, emb_table):
    return pl.pallas_call(
        _embed_body,
        grid=(N // BN,),
        in_specs=[
            pl.BlockSpec((BN, 1), lambda i: (i, 0)),
            pl.BlockSpec((NUM_TYPES, D), lambda i: (0, 0)),
        ],
        out_specs=pl.BlockSpec((BN, D), lambda i: (i, 0)),
        out_shape=jax.ShapeDtypeStruct((N, D), jnp.float32),
    )(seq_col, emb_table)


def _dense_body(x_ref, xp_ref, cd_ref, ws_ref, wmx_ref, wme_ref, bm_ref,
                bs_ref, o_ref):
    sseg = xp_ref[0] + xp_ref[1]
    cd = cd_ref[0] + cd_ref[1]
    h = _dot(x_ref[...], ws_ref[...]) + _dot(sseg, wmx_ref[...])
    h = h + cd[:, 0:1] * wme_ref[...] + cd[:, 1:2] * bm_ref[...] + bs_ref[...]
    o_ref[...] = _q(jnp.maximum(h, 0.0))


def _dense_layer(x, xp, cd, ws, wmx, wme_row, bm_row, bs_row):
    return pl.pallas_call(
        _dense_body,
        grid=(N // BN,),
        in_specs=[
            pl.BlockSpec((BN, D), lambda i: (i, 0)),
            pl.BlockSpec((NC, BN, D), lambda i: (0, i, 0)),
            pl.BlockSpec((NC, BN, 2), lambda i: (0, i, 0)),
            pl.BlockSpec((D, D), lambda i: (0, 0)),
            pl.BlockSpec((D, D), lambda i: (0, 0)),
            pl.BlockSpec((1, D), lambda i: (0, 0)),
            pl.BlockSpec((1, D), lambda i: (0, 0)),
            pl.BlockSpec((1, D), lambda i: (0, 0)),
        ],
        out_specs=pl.BlockSpec((BN, D), lambda i: (i, 0)),
        out_shape=jax.ShapeDtypeStruct((N, D), jnp.float32),
    )(x, xp, cd, ws, wmx, wme_row, bm_row, bs_row)


def _heads_body(x_ref, wo_ref, bo_ref, wh_ref, bh_ref, o_ref):
    emb = _dot(x_ref[...], wo_ref[...]) + bo_ref[...]
    h = _dot(_q(emb), wh_ref[...]) + bh_ref[...]
    col = lax.broadcasted_iota(jnp.int32, h.shape, 1)
    h = jnp.where(col == 2, jax.nn.sigmoid(h), h)
    o_ref[...] = h[:, :8]


def _heads(x, w_out, bo_row, wh, bh_row):
    return pl.pallas_call(
        _heads_body,
        grid=(N // BN,),
        in_specs=[
            pl.BlockSpec((BN, D), lambda i: (i, 0)),
            pl.BlockSpec((D, D), lambda i: (0, 0)),
            pl.BlockSpec((1, D), lambda i: (0, 0)),
            pl.BlockSpec((D, D), lambda i: (0, 0)),
            pl.BlockSpec((1, D), lambda i: (0, 0)),
        ],
        out_specs=pl.BlockSpec((BN, 8), lambda i: (i, 0)),
        out_shape=jax.ShapeDtypeStruct((N, 8), jnp.float32),
    )(x, w_out, bo_row, wh, bh_row)


def _segsum_edge_body(x_hbm, src_hbm, dst_hbm, ea_hbm, zr_hbm, zc_hbm,
                      xp_hbm, cp_hbm, dp_hbm,
                      sidx, didx, rows, ea_v, ones_v, sem, acc, cacc, dacc):
    c = lax.axis_index("c")
    s = lax.axis_index("s")
    wid = c * NS + s
    pltpu.sync_copy(zr_hbm, acc.at[pl.ds(s * RT, RT)])
    pltpu.sync_copy(zc_hbm, cacc.at[pl.ds(s * RT, RT)])
    pltpu.sync_copy(zc_hbm, dacc.at[pl.ds(s * RT, RT)])
    for j in range(CK // 16):
        ones_v[pl.ds(j * 16, 16)] = jnp.ones((16,), jnp.float32)
    plsc.subcore_barrier()

    base = wid * (EC * CK)

    def chunk(i, carry):
        off = base + i * CK
        pltpu.sync_copy(src_hbm.at[pl.ds(off, CK)], sidx)
        pltpu.sync_copy(dst_hbm.at[pl.ds(off, CK)], didx)
        pltpu.async_copy(x_hbm.at[sidx], rows, sem).wait()
        pltpu.sync_copy(rows, acc.at[didx], add=True)
        pltpu.sync_copy(ea_hbm.at[pl.ds(off, CK)], ea_v)
        pltpu.sync_copy(ea_v, cacc.at[didx], add=True)
        pltpu.sync_copy(ones_v, dacc.at[didx], add=True)
        return carry

    lax.fori_loop(0, EC, chunk, 0)
    plsc.subcore_barrier()
    pltpu.sync_copy(acc.at[pl.ds(s * RT, RT)], xp_hbm.at[c, pl.ds(s * RT, RT)])
    pltpu.sync_copy(cacc.at[pl.ds(s * RT, RT)],
                    cp_hbm.at[pl.ds(c * NPAD + s * RT, RT)])
    pltpu.sync_copy(dacc.at[pl.ds(s * RT, RT)],
                    dp_hbm.at[pl.ds(c * NPAD + s * RT, RT)])


_segsum_edge = pl.kernel(
    _segsum_edge_body,
    out_type=(
        jax.ShapeDtypeStruct((NC, NPAD, D), jnp.float32),
        jax.ShapeDtypeStruct((NC * NPAD,), jnp.float32),
        jax.ShapeDtypeStruct((NC * NPAD,), jnp.float32),
    ),
    mesh=_MESH,
    scratch_types=[
        pltpu.VMEM((CK,), jnp.int32),
        pltpu.VMEM((CK,), jnp.int32),
        pltpu.VMEM((CK, D), jnp.float32),
        pltpu.VMEM((CK,), jnp.float32),
        pltpu.VMEM((CK,), jnp.float32),
        pltpu.SemaphoreType.DMA,
        pltpu.VMEM_SHARED((NPAD, D), jnp.float32),
        pltpu.VMEM_SHARED((NPAD,), jnp.float32),
        pltpu.VMEM_SHARED((NPAD,), jnp.float32),
    ],
)


def _segsum_plain_body(x_hbm, src_hbm, dst_hbm, zr_hbm,
                       xp_hbm, sidx, didx, rows, sem, acc):
    c = lax.axis_index("c")
    s = lax.axis_index("s")
    wid = c * NS + s
    pltpu.sync_copy(zr_hbm, acc.at[pl.ds(s * RT, RT)])
    plsc.subcore_barrier()

    base = wid * (EC * CK)

    def chunk(i, carry):
        off = base + i * CK
        pltpu.sync_copy(src_hbm.at[pl.ds(off, CK)], sidx)
        pltpu.sync_copy(dst_hbm.at[pl.ds(off, CK)], didx)
        pltpu.async_copy(x_hbm.at[sidx], rows, sem).wait()
        pltpu.sync_copy(rows, acc.at[didx], add=True)
        return carry

    lax.fori_loop(0, EC, chunk, 0)
    plsc.subcore_barrier()
    pltpu.sync_copy(acc.at[pl.ds(s * RT, RT)], xp_hbm.at[c, pl.ds(s * RT, RT)])


_segsum_plain = pl.kernel(
    _segsum_plain_body,
    out_type=jax.ShapeDtypeStruct((NC, NPAD, D), jnp.float32),
    mesh=_MESH,
    scratch_types=[
        pltpu.VMEM((CK,), jnp.int32),
        pltpu.VMEM((CK,), jnp.int32),
        pltpu.VMEM((CK, D), jnp.float32),
        pltpu.SemaphoreType.DMA,
        pltpu.VMEM_SHARED((NPAD, D), jnp.float32),
    ],
)


def _edge_head_body(le_hbm, re_hbm, p0_hbm, p1_hbm, o_hbm,
                    i0, i1, a_v, b_v, o_v, sem):
    c = lax.axis_index("c")
    s = lax.axis_index("s")
    wid = c * NS + s
    base = wid * (PC * CK)

    def chunk(i, carry):
        off = base + i * CK
        pltpu.sync_copy(p0_hbm.at[pl.ds(off, CK)], i0)
        pltpu.sync_copy(p1_hbm.at[pl.ds(off, CK)], i1)
        pltpu.async_copy(le_hbm.at[i0], a_v, sem).wait()
        pltpu.async_copy(re_hbm.at[i1], b_v, sem).wait()
        for j in range(CK // 16):
            sl = pl.ds(j * 16, 16)
            z = a_v[sl] + b_v[sl]
            o_v[sl] = 1.0 / (1.0 + jnp.exp(-z))
        pltpu.sync_copy(o_v, o_hbm.at[pl.ds(off, CK)])
        return carry

    lax.fori_loop(0, PC, chunk, 0)


_edge_head = pl.kernel(
    _edge_head_body,
    out_type=jax.ShapeDtypeStruct((P_PAD,), jnp.float32),
    mesh=_MESH,
    scratch_types=[
        pltpu.VMEM((CK,), jnp.int32),
        pltpu.VMEM((CK,), jnp.int32),
        pltpu.VMEM((CK,), jnp.float32),
        pltpu.VMEM((CK,), jnp.float32),
        pltpu.VMEM((CK,), jnp.float32),
        pltpu.SemaphoreType.DMA,
    ],
)


def kernel(seq, nedge, edgeattr, pedge, emb_table,
           Wm1, bm1, Ws1, bs1, Wm2, bm2, Ws2, bs2,
           W_out, b_out, W_e, b_e, W_n, b_n):
    f32 = jnp.float32
    seq_col = seq.astype(jnp.int32).reshape(N, 1)
    epad = E_PAD - E
    srcp = jnp.concatenate([nedge[0].astype(jnp.int32),
                            jnp.zeros((epad,), jnp.int32)])
    dstp = jnp.concatenate([nedge[1].astype(jnp.int32),
                            jnp.full((epad,), N, jnp.int32)])
    eap = jnp.concatenate([_q(edgeattr[:, 0].astype(f32)),
                           jnp.zeros((epad,), f32)])
    ppad = P_PAD - P
    p0p = jnp.concatenate([pedge[0].astype(jnp.int32),
                           jnp.zeros((ppad,), jnp.int32)])
    p1p = jnp.concatenate([pedge[1].astype(jnp.int32),
                           jnp.zeros((ppad,), jnp.int32)])
    zr = jnp.zeros((RT, D), f32)
    zc = jnp.zeros((RT,), f32)

    x0 = _embed(seq_col, emb_table)
    xp1, cp, dp = _segsum_edge(x0, srcp, dstp, eap, zr, zc)
    cp = cp.reshape(NC, NPAD)
    dp = dp.reshape(NC, NPAD)
    cd = jnp.concatenate([cp[..., None], dp[..., None]], axis=-1)

    x1 = _dense_layer(x0, xp1, cd, _q(Ws1), _q(Wm1[:D]), _q(Wm1[D:D + 1]),
                      bm1[None, :], bs1[None, :])

    xp2 = _segsum_plain(x1, srcp, dstp, zr)

    x2 = _dense_layer(x1, xp2, cd, _q(Ws2), _q(Wm2[:D]), _q(Wm2[D:D + 1]),
                      bm2[None, :], bs2[None, :])

    wh = jnp.concatenate([W_e[:D], W_e[D:], W_n, jnp.zeros((D, 125), f32)],
                         axis=1)
    bh = jnp.concatenate([b_e, jnp.zeros((1,), f32), b_n,
                          jnp.zeros((125,), f32)])[None, :]
    lrp = _heads(x2, _q(W_out), b_out[None, :], _q(wh), bh)

    le = lrp[:, 0]
    re = lrp[:, 1]
    pred_node = lrp[:, 2:3]
    pe = _edge_head(le, re, p0p, p1p)
    pred_edge = pe[:P].reshape(P, 1)
    return (pred_edge, pred_node)
